# baseline (speedup 1.0000x reference)
import functools
import math

import jax
import jax.numpy as jnp
from jax import lax
from jax.experimental import pallas as pl
from jax.experimental.pallas import tpu as pltpu
from jax.experimental.pallas import tpu_sc as plsc

F32 = jnp.float32
I32 = jnp.int32
NW = 32
NTILE = 16
C = 128


def _round_up(x, m):
    return (x + m - 1) // m * m


CO = 64


@functools.cache
def _edge_pass(n_out_pad, e_pad, d):
    w_per = e_pad // (NW * C)
    mesh = plsc.VectorSubcoreMesh(core_axis_name="c", subcore_axis_name="s")
    rpt = n_out_pad // NTILE
    assert rpt % CO == 0

    @functools.partial(
        pl.kernel,
        name="edge_pass",
        out_type=jax.ShapeDtypeStruct((2, n_out_pad, d), F32),
        mesh=mesh,
        scratch_types=[
            pltpu.VMEM((C,), I32),
            pltpu.VMEM((C,), I32),
            pltpu.VMEM((C, d), F32),
            pltpu.VMEM((CO, d), F32),
            pltpu.VMEM_SHARED((n_out_pad, d), F32),
            pltpu.SemaphoreType.DMA,
        ],
    )
    def k(hs, srcp, dstp, out, src_v, dst_v, rows_v, stg_v, acc, sem):
        c = lax.axis_index("c")
        s = lax.axis_index("s")
        wid = c * NTILE + s

        def zfill(i, carry):
            stg_v[i // jnp.int32(d // 16),
                  pl.ds((i % jnp.int32(d // 16)) * 16, 16)] = (
                      jnp.zeros((16,), F32))
            return carry
        lax.fori_loop(jnp.int32(0), jnp.int32(CO * d // 16), zfill,
                      jnp.int32(0))

        def zcp(i, carry):
            pltpu.sync_copy(stg_v, acc.at[pl.ds(s * rpt + i * jnp.int32(CO),
                                                CO)])
            return carry
        lax.fori_loop(jnp.int32(0), jnp.int32(rpt // CO), zcp, jnp.int32(0))
        plsc.subcore_barrier()

        def body(w, carry):
            base = (wid * jnp.int32(w_per) + w) * jnp.int32(C)
            pltpu.sync_copy(srcp.at[pl.ds(base, C)], src_v)
            pltpu.sync_copy(dstp.at[pl.ds(base, C)], dst_v)
            pltpu.async_copy(hs.at[src_v], rows_v, sem).wait()
            pltpu.sync_copy(rows_v, acc.at[dst_v], add=True)
            return carry

        lax.fori_loop(jnp.int32(0), jnp.int32(w_per), body, jnp.int32(0))
        plsc.subcore_barrier()

        def ocp(i, carry):
            off = s * rpt + i * jnp.int32(CO)
            pltpu.sync_copy(acc.at[pl.ds(off, CO)], stg_v)
            pltpu.sync_copy(stg_v, out.at[c, pl.ds(off, CO)])
            return carry
        lax.fori_loop(jnp.int32(0), jnp.int32(rpt // CO), ocp, jnp.int32(0))

    return k


@functools.cache
def _deg_pass(n_out_pad, e_pad):
    w_per = e_pad // (NW * C)
    mesh = plsc.VectorSubcoreMesh(core_axis_name="c", subcore_axis_name="s")
    rpt = n_out_pad // NTILE

    @functools.partial(
        pl.kernel,
        name="deg_pass",
        out_type=jax.ShapeDtypeStruct((2 * n_out_pad,), F32),
        mesh=mesh,
        scratch_types=[
            pltpu.VMEM((C,), I32),
            pltpu.VMEM((C,), F32),
            pltpu.VMEM((rpt,), F32),
            pltpu.VMEM_SHARED((n_out_pad,), F32),
        ],
    )
    def k(dstp, out, dst_v, ones_v, stg_v, acc):
        c = lax.axis_index("c")
        s = lax.axis_index("s")
        wid = c * NTILE + s
        for i in range(C // 16):
            ones_v[pl.ds(i * 16, 16)] = jnp.full((16,), 1.0, F32)

        def zfill(i, carry):
            stg_v[pl.ds(i * 16, 16)] = jnp.zeros((16,), F32)
            return carry
        lax.fori_loop(jnp.int32(0), jnp.int32(rpt // 16), zfill, jnp.int32(0))
        pltpu.sync_copy(stg_v, acc.at[pl.ds(s * rpt, rpt)])
        plsc.subcore_barrier()

        def body(w, carry):
            base = (wid * jnp.int32(w_per) + w) * jnp.int32(C)
            pltpu.sync_copy(dstp.at[pl.ds(base, C)], dst_v)
            pltpu.sync_copy(ones_v, acc.at[dst_v], add=True)
            return carry

        lax.fori_loop(jnp.int32(0), jnp.int32(w_per), body, jnp.int32(0))
        plsc.subcore_barrier()
        pltpu.sync_copy(acc.at[pl.ds(s * rpt, rpt)], stg_v)
        pltpu.sync_copy(
            stg_v, out.at[pl.ds(c * jnp.int32(n_out_pad) + s * rpt, rpt)])

    return k


def _iota16():
    return lax.iota(I32, 16)


@functools.cache
def _map_edges(n_pad, k_pad, e_pad, kk, tbl):
    w_map = k_pad // (NTILE * C)
    w_per = e_pad // (NW * C)
    mesh = plsc.VectorSubcoreMesh(core_axis_name="c", subcore_axis_name="s")
    rpt = n_pad // NTILE
    ktrash = kk * kk

    @functools.partial(
        pl.kernel,
        name="map_edges",
        out_type=(jax.ShapeDtypeStruct((e_pad,), I32),
                  jax.ShapeDtypeStruct((e_pad,), I32),
                  jax.ShapeDtypeStruct((e_pad,), I32),
                  jax.ShapeDtypeStruct((tbl,), I32)),
        mesh=mesh,
        scratch_types=[
            pltpu.VMEM((C,), I32),
            pltpu.VMEM((C,), I32),
            pltpu.VMEM((C,), I32),
            pltpu.VMEM((C,), I32),
            pltpu.VMEM((C,), I32),
            pltpu.VMEM((C,), I32),
            pltpu.VMEM((rpt,), I32),
            pltpu.VMEM_SHARED((n_pad,), I32),
        ],
    )
    def k(idxp, srcp, dstp, s1o, d1o, keyo, tblo,
          a_v, b_v, s_v, d_v, key_v, eid_v, stg_v, newid):
        c = lax.axis_index("c")
        s = lax.axis_index("s")
        wid = c * NTILE + s

        def ifill(i, carry):
            stg_v[pl.ds(i * 16, 16)] = jnp.full((16,), -1, I32)
            return carry
        lax.fori_loop(jnp.int32(0), jnp.int32(rpt // 16), ifill, jnp.int32(0))
        pltpu.sync_copy(stg_v, newid.at[pl.ds(s * rpt, rpt)])
        plsc.subcore_barrier()

        def mbody(w, carry):
            base = (s * jnp.int32(w_map) + w) * jnp.int32(C)
            pltpu.sync_copy(idxp.at[pl.ds(base, C)], a_v)
            for j in range(C // 16):
                eid_v[pl.ds(j * 16, 16)] = base + jnp.int32(j * 16) + _iota16()
            pltpu.sync_copy(eid_v, newid.at[a_v])
            return carry
        lax.fori_loop(jnp.int32(0), jnp.int32(w_map), mbody, jnp.int32(0))
        plsc.subcore_barrier()

        def body(w, carry):
            base = (wid * jnp.int32(w_per) + w) * jnp.int32(C)
            pltpu.sync_copy(srcp.at[pl.ds(base, C)], a_v)
            pltpu.sync_copy(dstp.at[pl.ds(base, C)], b_v)
            pltpu.sync_copy(newid.at[a_v], s_v)
            pltpu.sync_copy(newid.at[b_v], d_v)
            for j in range(C // 16):
                sl = pl.ds(j * 16, 16)
                s16 = s_v[sl]
                d16 = d_v[sl]
                eid16 = base + jnp.int32(j * 16) + _iota16()
                m16 = (s16 >= 0) & (d16 >= 0)
                key16 = jnp.where(m16, s16 * jnp.int32(kk) + d16,
                                  jnp.int32(ktrash) + eid16)
                key_v[sl] = key16
                eid_v[sl] = eid16
            pltpu.sync_copy(s_v, s1o.at[pl.ds(base, C)])
            pltpu.sync_copy(d_v, d1o.at[pl.ds(base, C)])
            pltpu.sync_copy(key_v, keyo.at[pl.ds(base, C)])
            pltpu.sync_copy(eid_v, tblo.at[key_v])
            return carry
        lax.fori_loop(jnp.int32(0), jnp.int32(w_per), body, jnp.int32(0))

    return k


@functools.cache
def _finish_edges(pad_k, e_pad, kk):
    w_per = e_pad // (NW * C)
    mesh = plsc.VectorSubcoreMesh(core_axis_name="c", subcore_axis_name="s")
    rpt = pad_k // NTILE

    @functools.partial(
        pl.kernel,
        name="finish_edges",
        out_type=(jax.ShapeDtypeStruct((e_pad,), I32),
                  jax.ShapeDtypeStruct((e_pad,), I32),
                  jax.ShapeDtypeStruct((2 * pad_k,), F32)),
        mesh=mesh,
        scratch_types=[
            pltpu.VMEM((C,), I32),
            pltpu.VMEM((C,), I32),
            pltpu.VMEM((C,), I32),
            pltpu.VMEM((C,), I32),
            pltpu.VMEM((C,), I32),
            pltpu.VMEM((C,), I32),
            pltpu.VMEM((C,), F32),
            pltpu.VMEM((rpt,), F32),
            pltpu.VMEM_SHARED((pad_k,), F32),
            pltpu.SemaphoreType.DMA,
        ],
    )
    def k(s1, d1, key, tbl, srcfo, dstfo, dego,
          s_v, d_v, key_v, t_v, sf_v, df_v, ones_v, stg_v, acc, sem):
        c = lax.axis_index("c")
        s = lax.axis_index("s")
        wid = c * NTILE + s
        for i in range(C // 16):
            ones_v[pl.ds(i * 16, 16)] = jnp.full((16,), 1.0, F32)

        def zfill(i, carry):
            stg_v[pl.ds(i * 16, 16)] = jnp.zeros((16,), F32)
            return carry
        lax.fori_loop(jnp.int32(0), jnp.int32(rpt // 16), zfill, jnp.int32(0))
        pltpu.sync_copy(stg_v, acc.at[pl.ds(s * rpt, rpt)])
        plsc.subcore_barrier()

        def body(w, carry):
            base = (wid * jnp.int32(w_per) + w) * jnp.int32(C)
            pltpu.sync_copy(s1.at[pl.ds(base, C)], s_v)
            pltpu.sync_copy(d1.at[pl.ds(base, C)], d_v)
            pltpu.sync_copy(key.at[pl.ds(base, C)], key_v)
            pltpu.async_copy(tbl.at[key_v], t_v, sem).wait()
            for j in range(C // 16):
                sl = pl.ds(j * 16, 16)
                s16 = s_v[sl]
                d16 = d_v[sl]
                eid16 = base + jnp.int32(j * 16) + _iota16()
                ok = (s16 >= 0) & (d16 >= 0) & (t_v[sl] == eid16)
                sf_v[sl] = jnp.where(ok, s16, eid16 & 4095)
                df_v[sl] = jnp.where(ok, d16,
                                     jnp.int32(kk) + (eid16 & 1023))
            pltpu.sync_copy(sf_v, srcfo.at[pl.ds(base, C)])
            pltpu.sync_copy(df_v, dstfo.at[pl.ds(base, C)])
            pltpu.sync_copy(ones_v, acc.at[df_v], add=True)
            return carry
        lax.fori_loop(jnp.int32(0), jnp.int32(w_per), body, jnp.int32(0))
        plsc.subcore_barrier()
        pltpu.sync_copy(acc.at[pl.ds(s * rpt, rpt)], stg_v)
        pltpu.sync_copy(
            stg_v, dego.at[pl.ds(c * jnp.int32(pad_k) + s * rpt, rpt)])

    return k


@functools.cache
def _topk_thresh(npad, n, kk):
    nb = npad // 128

    def body(y_ref, thr_ref):
        y = y_ref[...]
        ib = pltpu.bitcast(y, jnp.int32)
        key = ib ^ ((ib >> 31) & jnp.int32(0x7FFFFFFF))
        rowi = lax.broadcasted_iota(jnp.int32, (nb, 128), 0)
        lanei = lax.broadcasted_iota(jnp.int32, (nb, 128), 1)
        key = jnp.where(rowi * 128 + lanei < n, key, jnp.int32(-2**31))

        v = jnp.int32(0)
        for b in range(31, -1, -1):
            vc = v | jnp.int32(-2**31 if b == 31 else 1 << b)
            cand = vc ^ jnp.int32(-2**31)
            cnt = jnp.sum((key >= cand).astype(F32), dtype=F32)
            v = jnp.where(cnt >= F32(kk), vc, v)
        t = v ^ jnp.int32(-2**31)
        thr_ref[...] = jnp.full((8, 128), t, jnp.int32)

    return pl.pallas_call(
        body, out_shape=jax.ShapeDtypeStruct((8, 128), jnp.int32))


@functools.cache
def _compact(npad, n, kk, k_pad):
    cpt = npad // NTILE
    nv = cpt // 16
    mesh = plsc.VectorSubcoreMesh(core_axis_name="c", subcore_axis_name="s")

    @functools.partial(
        pl.kernel,
        name="topk_compact",
        out_type=jax.ShapeDtypeStruct((k_pad + 2048,), I32),
        mesh=mesh,
        compiler_params=pltpu.CompilerParams(needs_layout_passes=False),
        scratch_types=[
            pltpu.VMEM((cpt,), F32),
            pltpu.VMEM((8, 128), I32),
            pltpu.VMEM((16,), I32),
            pltpu.VMEM((16,), I32),
            pltpu.VMEM((32,), I32),
            pltpu.VMEM((16,), I32),
            pltpu.VMEM((16,), I32),
            pltpu.VMEM((1152,), I32),
            pltpu.VMEM_SHARED((32,), I32),
        ],
    )
    def k(yh, thrh, idxo, y_v, thr_vm, si_v, cv_v, call_v, pos_v, val_v,
          pat_v, counts_sp):
        c = lax.axis_index("c")
        s = lax.axis_index("s")

        @pl.when(c == 0)
        def _():
            pltpu.sync_copy(yh.at[pl.ds(s * jnp.int32(cpt), cpt)], y_v)
            pltpu.sync_copy(thrh, thr_vm)
            thr = thr_vm[0, pl.ds(0, 16)]

            def mono(y16):
                ib = plsc.bitcast(y16, I32)
                return ib ^ ((ib >> 31) & jnp.int32(0x7FFFFFFF))

            def masks(j):
                y16 = y_v[pl.ds(j * jnp.int32(16), 16)]
                u = mono(y16)
                node = s * jnp.int32(cpt) + j * jnp.int32(16) + _iota16()
                msk = node < jnp.int32(n)
                return node, msk & (u > thr), msk & (u == thr)

            def p1(j, carry):
                csel, ctie = carry
                _, sel, tie = masks(j)
                return (csel + plsc.all_reduce_population_count(sel),
                        ctie + plsc.all_reduce_population_count(tie))

            z16 = jnp.zeros((16,), I32)
            csel, ctie = lax.fori_loop(jnp.int32(0), jnp.int32(nv), p1, (z16, z16))

            si_v[...] = jnp.full((16,), 0, I32) + s
            cv_v[...] = csel
            pltpu.sync_copy(cv_v, counts_sp.at[si_v])
            si_v[...] = jnp.full((16,), 16, I32) + s
            cv_v[...] = ctie
            pltpu.sync_copy(cv_v, counts_sp.at[si_v])
            plsc.subcore_barrier()
            pltpu.sync_copy(counts_sp, call_v)
            selc = call_v[pl.ds(0, 16)]
            tiec = call_v[pl.ds(16, 16)]
            inc = plsc.cumsum(selc)
            tin = plsc.cumsum(tiec)
            lane_s = jnp.full((16,), 0, I32) + s
            cv_v[...] = inc - selc
            soff = plsc.load_gather(cv_v, [lane_s])
            cv_v[...] = tin - tiec
            toff = plsc.load_gather(cv_v, [lane_s])
            cv_v[...] = inc
            g_tot = plsc.load_gather(cv_v, [jnp.full((16,), 15, I32)])
            rneed = jnp.full((16,), kk, I32) - g_tot

            def p2(j, carry):
                so, to = carry
                node, sel, tie = masks(j)
                cs = plsc.cumsum(jnp.where(sel, jnp.int32(1), jnp.int32(0)))
                ct = plsc.cumsum(jnp.where(tie, jnp.int32(1), jnp.int32(0)))
                grank = to + ct - 1
                acc = tie & (grank < rneed)
                trash = jnp.full((16,), k_pad, I32) + s * jnp.int32(64) + j
                pos = jnp.where(sel, so + cs - 1,
                                jnp.where(acc, g_tot + grank, trash))
                pos_v[...] = pos
                val_v[...] = node
                pltpu.sync_copy(val_v, idxo.at[pos_v])
                return (so + plsc.all_reduce_population_count(sel),
                        to + plsc.all_reduce_population_count(tie))

            lax.fori_loop(jnp.int32(0), jnp.int32(nv), p2, (soff, toff))

            @pl.when(s == 0)
            def _():
                def pf(i, carry):
                    t16 = i * jnp.int32(16) + _iota16()
                    pat_v[pl.ds(i * jnp.int32(16), 16)] = jnp.int32(n + 64) + (t16 & 63)
                    return carry
                lax.fori_loop(jnp.int32(0), jnp.int32((k_pad - kk) // 16 + 1), pf, jnp.int32(0))
                pltpu.sync_copy(pat_v.at[pl.ds(0, k_pad - kk)],
                                idxo.at[pl.ds(kk, k_pad - kk)])

    return k


@functools.cache
def _gather_rows(m_rows, d):
    cw = 64
    w_per = m_rows // (NW * cw)
    mesh = plsc.VectorSubcoreMesh(core_axis_name="c", subcore_axis_name="s")

    @functools.partial(
        pl.kernel,
        name="gather_rows",
        out_type=jax.ShapeDtypeStruct((m_rows, d), F32),
        mesh=mesh,
        scratch_types=[
            pltpu.VMEM((cw,), I32),
            pltpu.VMEM((cw, d), F32),
            pltpu.SemaphoreType.DMA,
        ],
    )
    def k(hs, idxh, out, i_v, rows_v, sem):
        c = lax.axis_index("c")
        s = lax.axis_index("s")
        wid = c * NTILE + s

        def body(w, carry):
            base = (wid * jnp.int32(w_per) + w) * jnp.int32(cw)
            pltpu.sync_copy(idxh.at[pl.ds(base, cw)], i_v)
            pltpu.async_copy(hs.at[i_v], rows_v, sem).wait()
            pltpu.sync_copy(rows_v, out.at[pl.ds(base, cw)])
            return carry

        lax.fori_loop(jnp.int32(0), jnp.int32(w_per), body, jnp.int32(0))

    return k


def _pad_edges(src, dst, n_in, n_out):
    e = src.shape[0]
    e_pad = _round_up(e, NW * C)
    pad = e_pad - e
    i = jnp.arange(pad, dtype=I32)
    src_p = jnp.concatenate([src, i % jnp.int32(n_in)])
    dst_p = jnp.concatenate([dst, jnp.int32(n_out) + (i % 64)])
    return src_p, dst_p, e_pad


def _sinus_row(t, dim):
    half = dim // 2
    cst = math.log(10000.0) / (half - 1)
    freqs = jnp.exp(jnp.arange(half, dtype=F32) * (-cst))
    e = t[0].astype(F32) * freqs
    return jnp.concatenate([jnp.sin(e), jnp.cos(e)])


def kernel(noised_data, t, edge_index, W_in, b_in, W0, b0, W1, b1, W2, b2,
           p_w, p_b, gamma, beta, W_fc, b_fc):
    n = noised_data.shape[1]
    kk = n // 2
    d = W0.shape[1]
    ei = edge_index.astype(I32)
    src0, dst0 = ei[0], ei[1]
    e = src0.shape[0]

    pad_n = _round_up(n + 64, 1024)
    pad_k = _round_up(kk + 64, 1024)

    src0p, dst0p, e_pad = _pad_edges(src0, dst0, n, n)

    dp = _deg_pass(pad_n, e_pad)(dst0p)
    deg0 = dp[:n] + dp[pad_n:pad_n + n] + 1.0
    r0 = lax.rsqrt(deg0)

    x = noised_data[0] @ W_in + b_in
    temb = _sinus_row(t, d)
    const0 = temb @ W0[W_in.shape[1]:]
    H0 = x @ W0[: W_in.shape[1]] + const0
    Hs0 = H0 * r0[:, None]

    S = _edge_pass(pad_n, e_pad, d)(Hs0, src0p, dst0p)
    h0 = jax.nn.relu(r0[:, None] * (S[0, :n] + S[1, :n] + Hs0) + b0)

    k_pad = _round_up(kk, NTILE * C)
    y = (h0 @ p_w)[:, 0] + p_b[0]
    ypad = jnp.pad(y, (0, pad_n - n))
    thr = _topk_thresh(pad_n, n, kk)(ypad.reshape(pad_n // 128, 128))
    idxfull = _compact(pad_n, n, kk, k_pad)(ypad, thr)
    idx32 = idxfull[:kk]
    g0 = h0 * jax.nn.sigmoid(y)[:, None]
    g0p = jnp.pad(g0, ((0, pad_n - n), (0, 0)))
    x1 = _gather_rows(k_pad, d)(g0p, idxfull)[:kk]

    tbl = kk * kk + e_pad
    s1a, d1a, keya, tbla = _map_edges(pad_n, k_pad, e_pad, kk, tbl)(
        idxfull, src0p, dst0p)
    s1p, d1p, dego1 = _finish_edges(pad_k, e_pad, kk)(s1a, d1a, keya, tbla)
    deg1 = dego1[:kk] + dego1[pad_k:pad_k + kk] + 1.0
    r1 = lax.rsqrt(deg1)
    H1 = x1 @ W1
    Hs1 = H1 * r1[:, None]
    S1 = _edge_pass(pad_k, e_pad, d)(Hs1, s1p, d1p)
    h1 = jax.nn.relu(r1[:, None] * (S1[0, :kk] + S1[1, :kk] + Hs1) + b1)

    A = h0 @ W2
    B = h1 @ W2
    usrc, udst, ue_pad = _pad_edges(jnp.arange(kk, dtype=I32), idx32, kk, n)
    SU = _edge_pass(pad_n, ue_pad, d)(B, usrc, udst)
    U2 = SU[0, :n] + SU[1, :n] + A
    Hs2 = U2 * r0[:, None]

    S2 = _edge_pass(pad_n, e_pad, d)(Hs2, src0p, dst0p)
    h2 = jax.nn.relu(r0[:, None] * (S2[0, :n] + S2[1, :n] + Hs2) + b2)

    mu = jnp.mean(h2, axis=-1, keepdims=True)
    var = jnp.mean((h2 - mu) ** 2, axis=-1, keepdims=True)
    h2 = (h2 - mu) / jnp.sqrt(var + 1e-5) * gamma + beta
    return (h2 @ W_fc + b_fc)[None, ...]


# speedup vs baseline: 1.0042x; 1.0042x over previous
import functools
import math

import jax
import jax.numpy as jnp
from jax import lax
from jax.experimental import pallas as pl
from jax.experimental.pallas import tpu as pltpu
from jax.experimental.pallas import tpu_sc as plsc

F32 = jnp.float32
I32 = jnp.int32
NW = 32
NTILE = 16
C = 128


def _round_up(x, m):
    return (x + m - 1) // m * m


CO = 64


@functools.cache
def _edge_pass(n_out_pad, e_pad, d):
    w_per = e_pad // (NW * C)
    mesh = plsc.VectorSubcoreMesh(core_axis_name="c", subcore_axis_name="s")
    rpt = n_out_pad // NTILE
    assert rpt % CO == 0

    @functools.partial(
        pl.kernel,
        name="edge_pass",
        out_type=jax.ShapeDtypeStruct((2, n_out_pad, d), F32),
        mesh=mesh,
        scratch_types=[
            pltpu.VMEM((C,), I32),
            pltpu.VMEM((C,), I32),
            pltpu.VMEM((C, d), F32),
            pltpu.VMEM((CO, d), F32),
            pltpu.VMEM_SHARED((n_out_pad, d), F32),
            pltpu.SemaphoreType.DMA,
        ],
    )
    def k(hs, srcp, dstp, out, src_v, dst_v, rows_v, stg_v, acc, sem):
        c = lax.axis_index("c")
        s = lax.axis_index("s")
        wid = c * NTILE + s

        def zfill(i, carry):
            stg_v[i // jnp.int32(d // 16),
                  pl.ds((i % jnp.int32(d // 16)) * 16, 16)] = (
                      jnp.zeros((16,), F32))
            return carry
        lax.fori_loop(jnp.int32(0), jnp.int32(CO * d // 16), zfill,
                      jnp.int32(0))

        def zcp(i, carry):
            pltpu.sync_copy(stg_v, acc.at[pl.ds(s * rpt + i * jnp.int32(CO),
                                                CO)])
            return carry
        lax.fori_loop(jnp.int32(0), jnp.int32(rpt // CO), zcp, jnp.int32(0))
        plsc.subcore_barrier()

        def body(w, carry):
            base = (wid * jnp.int32(w_per) + w) * jnp.int32(C)
            pltpu.sync_copy(srcp.at[pl.ds(base, C)], src_v)
            pltpu.sync_copy(dstp.at[pl.ds(base, C)], dst_v)
            pltpu.async_copy(hs.at[src_v], rows_v, sem).wait()
            pltpu.sync_copy(rows_v, acc.at[dst_v], add=True)
            return carry

        lax.fori_loop(jnp.int32(0), jnp.int32(w_per), body, jnp.int32(0))
        plsc.subcore_barrier()

        def ocp(i, carry):
            off = s * rpt + i * jnp.int32(CO)
            pltpu.sync_copy(acc.at[pl.ds(off, CO)], stg_v)
            pltpu.sync_copy(stg_v, out.at[c, pl.ds(off, CO)])
            return carry
        lax.fori_loop(jnp.int32(0), jnp.int32(rpt // CO), ocp, jnp.int32(0))

    return k


@functools.cache
def _deg_pass(n_out_pad, e_pad):
    w_per = e_pad // (NW * C)
    mesh = plsc.VectorSubcoreMesh(core_axis_name="c", subcore_axis_name="s")
    rpt = n_out_pad // NTILE

    @functools.partial(
        pl.kernel,
        name="deg_pass",
        out_type=jax.ShapeDtypeStruct((2 * n_out_pad,), F32),
        mesh=mesh,
        scratch_types=[
            pltpu.VMEM((C,), I32),
            pltpu.VMEM((C,), F32),
            pltpu.VMEM((rpt,), F32),
            pltpu.VMEM_SHARED((n_out_pad,), F32),
        ],
    )
    def k(dstp, out, dst_v, ones_v, stg_v, acc):
        c = lax.axis_index("c")
        s = lax.axis_index("s")
        wid = c * NTILE + s
        for i in range(C // 16):
            ones_v[pl.ds(i * 16, 16)] = jnp.full((16,), 1.0, F32)

        def zfill(i, carry):
            stg_v[pl.ds(i * 16, 16)] = jnp.zeros((16,), F32)
            return carry
        lax.fori_loop(jnp.int32(0), jnp.int32(rpt // 16), zfill, jnp.int32(0))
        pltpu.sync_copy(stg_v, acc.at[pl.ds(s * rpt, rpt)])
        plsc.subcore_barrier()

        def body(w, carry):
            base = (wid * jnp.int32(w_per) + w) * jnp.int32(C)
            pltpu.sync_copy(dstp.at[pl.ds(base, C)], dst_v)
            pltpu.sync_copy(ones_v, acc.at[dst_v], add=True)
            return carry

        lax.fori_loop(jnp.int32(0), jnp.int32(w_per), body, jnp.int32(0))
        plsc.subcore_barrier()
        pltpu.sync_copy(acc.at[pl.ds(s * rpt, rpt)], stg_v)
        pltpu.sync_copy(
            stg_v, out.at[pl.ds(c * jnp.int32(n_out_pad) + s * rpt, rpt)])

    return k


def _iota16():
    return lax.iota(I32, 16)


@functools.cache
def _map_edges(n_pad, k_pad, e_pad, kk, tbl):
    w_map = k_pad // (NTILE * C)
    w_per = e_pad // (NW * C)
    mesh = plsc.VectorSubcoreMesh(core_axis_name="c", subcore_axis_name="s")
    rpt = n_pad // NTILE
    ktrash = kk * kk

    @functools.partial(
        pl.kernel,
        name="map_edges",
        out_type=(jax.ShapeDtypeStruct((e_pad,), I32),
                  jax.ShapeDtypeStruct((e_pad,), I32),
                  jax.ShapeDtypeStruct((e_pad,), I32),
                  jax.ShapeDtypeStruct((tbl,), I32)),
        mesh=mesh,
        scratch_types=[
            pltpu.VMEM((C,), I32),
            pltpu.VMEM((C,), I32),
            pltpu.VMEM((C,), I32),
            pltpu.VMEM((C,), I32),
            pltpu.VMEM((C,), I32),
            pltpu.VMEM((C,), I32),
            pltpu.VMEM((rpt,), I32),
            pltpu.VMEM_SHARED((n_pad,), I32),
        ],
    )
    def k(idxp, srcp, dstp, s1o, d1o, keyo, tblo,
          a_v, b_v, s_v, d_v, key_v, eid_v, stg_v, newid):
        c = lax.axis_index("c")
        s = lax.axis_index("s")
        wid = c * NTILE + s

        def ifill(i, carry):
            stg_v[pl.ds(i * 16, 16)] = jnp.full((16,), -1, I32)
            return carry
        lax.fori_loop(jnp.int32(0), jnp.int32(rpt // 16), ifill, jnp.int32(0))
        pltpu.sync_copy(stg_v, newid.at[pl.ds(s * rpt, rpt)])
        plsc.subcore_barrier()

        def mbody(w, carry):
            base = (s * jnp.int32(w_map) + w) * jnp.int32(C)
            pltpu.sync_copy(idxp.at[pl.ds(base, C)], a_v)
            for j in range(C // 16):
                eid_v[pl.ds(j * 16, 16)] = base + jnp.int32(j * 16) + _iota16()
            pltpu.sync_copy(eid_v, newid.at[a_v])
            return carry
        lax.fori_loop(jnp.int32(0), jnp.int32(w_map), mbody, jnp.int32(0))
        plsc.subcore_barrier()

        def body(w, carry):
            base = (wid * jnp.int32(w_per) + w) * jnp.int32(C)
            pltpu.sync_copy(srcp.at[pl.ds(base, C)], a_v)
            pltpu.sync_copy(dstp.at[pl.ds(base, C)], b_v)
            pltpu.sync_copy(newid.at[a_v], s_v)
            pltpu.sync_copy(newid.at[b_v], d_v)
            for j in range(C // 16):
                sl = pl.ds(j * 16, 16)
                s16 = s_v[sl]
                d16 = d_v[sl]
                eid16 = base + jnp.int32(j * 16) + _iota16()
                m16 = (s16 >= 0) & (d16 >= 0)
                key16 = jnp.where(m16, s16 * jnp.int32(kk) + d16,
                                  jnp.int32(ktrash) + eid16)
                key_v[sl] = key16
                eid_v[sl] = eid16
            pltpu.sync_copy(s_v, s1o.at[pl.ds(base, C)])
            pltpu.sync_copy(d_v, d1o.at[pl.ds(base, C)])
            pltpu.sync_copy(key_v, keyo.at[pl.ds(base, C)])
            pltpu.sync_copy(eid_v, tblo.at[key_v])
            return carry
        lax.fori_loop(jnp.int32(0), jnp.int32(w_per), body, jnp.int32(0))

    return k


@functools.cache
def _finish_edges(pad_k, e_pad, kk):
    w_per = e_pad // (NW * C)
    mesh = plsc.VectorSubcoreMesh(core_axis_name="c", subcore_axis_name="s")
    rpt = pad_k // NTILE

    @functools.partial(
        pl.kernel,
        name="finish_edges",
        out_type=(jax.ShapeDtypeStruct((e_pad,), I32),
                  jax.ShapeDtypeStruct((e_pad,), I32),
                  jax.ShapeDtypeStruct((2 * pad_k,), F32)),
        mesh=mesh,
        scratch_types=[
            pltpu.VMEM((C,), I32),
            pltpu.VMEM((C,), I32),
            pltpu.VMEM((C,), I32),
            pltpu.VMEM((C,), I32),
            pltpu.VMEM((C,), I32),
            pltpu.VMEM((C,), I32),
            pltpu.VMEM((C,), F32),
            pltpu.VMEM((rpt,), F32),
            pltpu.VMEM_SHARED((pad_k,), F32),
            pltpu.SemaphoreType.DMA,
        ],
    )
    def k(s1, d1, key, tbl, srcfo, dstfo, dego,
          s_v, d_v, key_v, t_v, sf_v, df_v, ones_v, stg_v, acc, sem):
        c = lax.axis_index("c")
        s = lax.axis_index("s")
        wid = c * NTILE + s
        for i in range(C // 16):
            ones_v[pl.ds(i * 16, 16)] = jnp.full((16,), 1.0, F32)

        def zfill(i, carry):
            stg_v[pl.ds(i * 16, 16)] = jnp.zeros((16,), F32)
            return carry
        lax.fori_loop(jnp.int32(0), jnp.int32(rpt // 16), zfill, jnp.int32(0))
        pltpu.sync_copy(stg_v, acc.at[pl.ds(s * rpt, rpt)])
        plsc.subcore_barrier()

        def body(w, carry):
            base = (wid * jnp.int32(w_per) + w) * jnp.int32(C)
            pltpu.sync_copy(s1.at[pl.ds(base, C)], s_v)
            pltpu.sync_copy(d1.at[pl.ds(base, C)], d_v)
            pltpu.sync_copy(key.at[pl.ds(base, C)], key_v)
            pltpu.async_copy(tbl.at[key_v], t_v, sem).wait()
            for j in range(C // 16):
                sl = pl.ds(j * 16, 16)
                s16 = s_v[sl]
                d16 = d_v[sl]
                eid16 = base + jnp.int32(j * 16) + _iota16()
                ok = (s16 >= 0) & (d16 >= 0) & (t_v[sl] == eid16)
                sf_v[sl] = jnp.where(ok, s16, eid16 & 4095)
                df_v[sl] = jnp.where(ok, d16,
                                     jnp.int32(kk) + (eid16 & 1023))
            pltpu.sync_copy(sf_v, srcfo.at[pl.ds(base, C)])
            pltpu.sync_copy(df_v, dstfo.at[pl.ds(base, C)])
            pltpu.sync_copy(ones_v, acc.at[df_v], add=True)
            return carry
        lax.fori_loop(jnp.int32(0), jnp.int32(w_per), body, jnp.int32(0))
        plsc.subcore_barrier()
        pltpu.sync_copy(acc.at[pl.ds(s * rpt, rpt)], stg_v)
        pltpu.sync_copy(
            stg_v, dego.at[pl.ds(c * jnp.int32(pad_k) + s * rpt, rpt)])

    return k


@functools.cache
def _topk_thresh(npad, n, kk):
    nb = npad // 128

    def body(y_ref, thr_ref):
        y = y_ref[...]
        ib = pltpu.bitcast(y, jnp.int32)
        key = ib ^ ((ib >> 31) & jnp.int32(0x7FFFFFFF))
        rowi = lax.broadcasted_iota(jnp.int32, (nb, 128), 0)
        lanei = lax.broadcasted_iota(jnp.int32, (nb, 128), 1)
        key = jnp.where(rowi * 128 + lanei < n, key, jnp.int32(-2**31))

        v = jnp.int32(0)
        for b in range(31, -1, -1):
            vc = v | jnp.int32(-2**31 if b == 31 else 1 << b)
            cand = vc ^ jnp.int32(-2**31)
            cnt = jnp.sum((key >= cand).astype(F32), dtype=F32)
            v = jnp.where(cnt >= F32(kk), vc, v)
        t = v ^ jnp.int32(-2**31)
        thr_ref[...] = jnp.full((8, 128), t, jnp.int32)

    return pl.pallas_call(
        body, out_shape=jax.ShapeDtypeStruct((8, 128), jnp.int32))


@functools.cache
def _compact(npad, n, kk, k_pad):
    cpt = npad // NTILE
    nv = cpt // 16
    mesh = plsc.VectorSubcoreMesh(core_axis_name="c", subcore_axis_name="s")

    @functools.partial(
        pl.kernel,
        name="topk_compact",
        out_type=jax.ShapeDtypeStruct((k_pad + 2048,), I32),
        mesh=mesh,
        compiler_params=pltpu.CompilerParams(needs_layout_passes=False),
        scratch_types=[
            pltpu.VMEM((cpt,), F32),
            pltpu.VMEM((8, 128), I32),
            pltpu.VMEM((16,), I32),
            pltpu.VMEM((16,), I32),
            pltpu.VMEM((32,), I32),
            pltpu.VMEM((16,), I32),
            pltpu.VMEM((16,), I32),
            pltpu.VMEM((1152,), I32),
            pltpu.VMEM_SHARED((32,), I32),
        ],
    )
    def k(yh, thrh, idxo, y_v, thr_vm, si_v, cv_v, call_v, pos_v, val_v,
          pat_v, counts_sp):
        c = lax.axis_index("c")
        s = lax.axis_index("s")

        @pl.when(c == 0)
        def _():
            pltpu.sync_copy(yh.at[pl.ds(s * jnp.int32(cpt), cpt)], y_v)
            pltpu.sync_copy(thrh, thr_vm)
            thr = thr_vm[0, pl.ds(0, 16)]

            def mono(y16):
                ib = plsc.bitcast(y16, I32)
                return ib ^ ((ib >> 31) & jnp.int32(0x7FFFFFFF))

            def masks(j):
                y16 = y_v[pl.ds(j * jnp.int32(16), 16)]
                u = mono(y16)
                node = s * jnp.int32(cpt) + j * jnp.int32(16) + _iota16()
                msk = node < jnp.int32(n)
                return node, msk & (u > thr), msk & (u == thr)

            def p1(j, carry):
                csel, ctie = carry
                _, sel, tie = masks(j)
                return (csel + plsc.all_reduce_population_count(sel),
                        ctie + plsc.all_reduce_population_count(tie))

            z16 = jnp.zeros((16,), I32)
            csel, ctie = lax.fori_loop(jnp.int32(0), jnp.int32(nv), p1, (z16, z16))

            si_v[...] = jnp.full((16,), 0, I32) + s
            cv_v[...] = csel
            pltpu.sync_copy(cv_v, counts_sp.at[si_v])
            si_v[...] = jnp.full((16,), 16, I32) + s
            cv_v[...] = ctie
            pltpu.sync_copy(cv_v, counts_sp.at[si_v])
            plsc.subcore_barrier()
            pltpu.sync_copy(counts_sp, call_v)
            selc = call_v[pl.ds(0, 16)]
            tiec = call_v[pl.ds(16, 16)]
            inc = plsc.cumsum(selc)
            tin = plsc.cumsum(tiec)
            lane_s = jnp.full((16,), 0, I32) + s
            cv_v[...] = inc - selc
            soff = plsc.load_gather(cv_v, [lane_s])
            cv_v[...] = tin - tiec
            toff = plsc.load_gather(cv_v, [lane_s])
            cv_v[...] = inc
            g_tot = plsc.load_gather(cv_v, [jnp.full((16,), 15, I32)])
            rneed = jnp.full((16,), kk, I32) - g_tot

            def p2(j, carry):
                so, to = carry
                node, sel, tie = masks(j)
                cs = plsc.cumsum(jnp.where(sel, jnp.int32(1), jnp.int32(0)))
                ct = plsc.cumsum(jnp.where(tie, jnp.int32(1), jnp.int32(0)))
                grank = to + ct - 1
                acc = tie & (grank < rneed)
                trash = jnp.full((16,), k_pad, I32) + s * jnp.int32(64) + j
                pos = jnp.where(sel, so + cs - 1,
                                jnp.where(acc, g_tot + grank, trash))
                pos_v[...] = pos
                val_v[...] = node
                pltpu.sync_copy(val_v, idxo.at[pos_v])
                return (so + plsc.all_reduce_population_count(sel),
                        to + plsc.all_reduce_population_count(tie))

            lax.fori_loop(jnp.int32(0), jnp.int32(nv), p2, (soff, toff))

            @pl.when(s == 0)
            def _():
                def pf(i, carry):
                    t16 = i * jnp.int32(16) + _iota16()
                    pat_v[pl.ds(i * jnp.int32(16), 16)] = jnp.int32(n + 64) + (t16 & 63)
                    return carry
                lax.fori_loop(jnp.int32(0), jnp.int32((k_pad - kk) // 16 + 1), pf, jnp.int32(0))
                pltpu.sync_copy(pat_v.at[pl.ds(0, k_pad - kk)],
                                idxo.at[pl.ds(kk, k_pad - kk)])

    return k


@functools.cache
def _gather_rows(m_rows, d):
    cw = 64
    w_per = m_rows // (NW * cw)
    mesh = plsc.VectorSubcoreMesh(core_axis_name="c", subcore_axis_name="s")

    @functools.partial(
        pl.kernel,
        name="gather_rows",
        out_type=jax.ShapeDtypeStruct((m_rows, d), F32),
        mesh=mesh,
        scratch_types=[
            pltpu.VMEM((cw,), I32),
            pltpu.VMEM((cw, d), F32),
            pltpu.SemaphoreType.DMA,
        ],
    )
    def k(hs, idxh, out, i_v, rows_v, sem):
        c = lax.axis_index("c")
        s = lax.axis_index("s")
        wid = c * NTILE + s

        def body(w, carry):
            base = (wid * jnp.int32(w_per) + w) * jnp.int32(cw)
            pltpu.sync_copy(idxh.at[pl.ds(base, cw)], i_v)
            pltpu.async_copy(hs.at[i_v], rows_v, sem).wait()
            pltpu.sync_copy(rows_v, out.at[pl.ds(base, cw)])
            return carry

        lax.fori_loop(jnp.int32(0), jnp.int32(w_per), body, jnp.int32(0))

    return k


def _i32(v):
    return lax.convert_element_type(v, jnp.int32)


def _rep(shape):
    return lambda i: tuple(_i32(i) * 0 for _ in shape)


@functools.cache
def _front(npr, br, din):
    def body(x_ref, wi_ref, bi_ref, w0_ref, c0_ref, r_ref, o_ref):
        h = jnp.dot(x_ref[...], wi_ref[...],
                    preferred_element_type=F32) + bi_ref[...]
        h = jnp.dot(h, w0_ref[...], preferred_element_type=F32) + c0_ref[...]
        o_ref[...] = h * r_ref[...]

    return pl.pallas_call(
        body,
        grid=(npr // br,),
        in_specs=[pl.BlockSpec((br, din), lambda i: (_i32(i), _i32(i) * 0)),
                  pl.BlockSpec((din, 128), _rep((0, 0))),
                  pl.BlockSpec((1, 128), _rep((0, 0))),
                  pl.BlockSpec((128, 128), _rep((0, 0))),
                  pl.BlockSpec((1, 128), _rep((0, 0))),
                  pl.BlockSpec((br, 1), lambda i: (_i32(i), _i32(i) * 0))],
        out_specs=pl.BlockSpec((br, 128), lambda i: (_i32(i), _i32(i) * 0)),
        out_shape=jax.ShapeDtypeStruct((npr, 128), F32))


@functools.cache
def _post0(npr, br):
    def body(s_ref, hs_ref, r_ref, b_ref, pw_ref, pb_ref, w1_ref, w2_ref,
             y_ref, q_ref, a_ref):
        s = s_ref[...]
        h = jax.nn.relu(r_ref[...] * (s[0] + s[1] + hs_ref[...]) + b_ref[...])
        y = jnp.dot(h, pw_ref[...], preferred_element_type=F32) + pb_ref[0, 0]
        g = h * (1.0 / (1.0 + jnp.exp(-y)))
        y_ref[...] = y
        q_ref[...] = jnp.dot(g, w1_ref[...], preferred_element_type=F32)
        a_ref[...] = jnp.dot(h, w2_ref[...], preferred_element_type=F32)

    return pl.pallas_call(
        body,
        grid=(npr // br,),
        in_specs=[pl.BlockSpec((2, br, 128), lambda i: (_i32(i) * 0, _i32(i), _i32(i) * 0)),
                  pl.BlockSpec((br, 128), lambda i: (_i32(i), _i32(i) * 0)),
                  pl.BlockSpec((br, 1), lambda i: (_i32(i), _i32(i) * 0)),
                  pl.BlockSpec((1, 128), _rep((0, 0))),
                  pl.BlockSpec((128, 1), _rep((0, 0))),
                  pl.BlockSpec(memory_space=pltpu.SMEM),
                  pl.BlockSpec((128, 128), _rep((0, 0))),
                  pl.BlockSpec((128, 128), _rep((0, 0)))],
        out_specs=[pl.BlockSpec((br, 1), lambda i: (_i32(i), _i32(i) * 0)),
                   pl.BlockSpec((br, 128), lambda i: (_i32(i), _i32(i) * 0)),
                   pl.BlockSpec((br, 128), lambda i: (_i32(i), _i32(i) * 0))],
        out_shape=[jax.ShapeDtypeStruct((npr, 1), F32),
                   jax.ShapeDtypeStruct((npr, 128), F32),
                   jax.ShapeDtypeStruct((npr, 128), F32)])


@functools.cache
def _post1(npr, br):
    def body(s_ref, hs_ref, r_ref, b_ref, w2_ref, o_ref):
        s = s_ref[...]
        h = jax.nn.relu(r_ref[...] * (s[0] + s[1] + hs_ref[...]) + b_ref[...])
        o_ref[...] = jnp.dot(h, w2_ref[...], preferred_element_type=F32)

    return pl.pallas_call(
        body,
        grid=(npr // br,),
        in_specs=[pl.BlockSpec((2, br, 128), lambda i: (_i32(i) * 0, _i32(i), _i32(i) * 0)),
                  pl.BlockSpec((br, 128), lambda i: (_i32(i), _i32(i) * 0)),
                  pl.BlockSpec((br, 1), lambda i: (_i32(i), _i32(i) * 0)),
                  pl.BlockSpec((1, 128), _rep((0, 0))),
                  pl.BlockSpec((128, 128), _rep((0, 0)))],
        out_specs=pl.BlockSpec((br, 128), lambda i: (_i32(i), _i32(i) * 0)),
        out_shape=jax.ShapeDtypeStruct((npr, 128), F32))


@functools.cache
def _post2(npr, br, dout):
    def body(s_ref, hs_ref, r_ref, b_ref, g_ref, be_ref, wf_ref, bf_ref,
             o_ref):
        s = s_ref[...]
        h = jax.nn.relu(r_ref[...] * (s[0] + s[1] + hs_ref[...]) + b_ref[...])
        mu = jnp.mean(h, axis=-1, keepdims=True)
        var = jnp.mean((h - mu) ** 2, axis=-1, keepdims=True)
        ln = (h - mu) / jnp.sqrt(var + 1e-5) * g_ref[...] + be_ref[...]
        o_ref[...] = jnp.dot(ln, wf_ref[...],
                             preferred_element_type=F32) + bf_ref[...]

    return pl.pallas_call(
        body,
        grid=(npr // br,),
        in_specs=[pl.BlockSpec((2, br, 128), lambda i: (_i32(i) * 0, _i32(i), _i32(i) * 0)),
                  pl.BlockSpec((br, 128), lambda i: (_i32(i), _i32(i) * 0)),
                  pl.BlockSpec((br, 1), lambda i: (_i32(i), _i32(i) * 0)),
                  pl.BlockSpec((1, 128), _rep((0, 0))),
                  pl.BlockSpec((1, 128), _rep((0, 0))),
                  pl.BlockSpec((1, 128), _rep((0, 0))),
                  pl.BlockSpec((128, dout), _rep((0, 0))),
                  pl.BlockSpec((1, dout), _rep((0, 0)))],
        out_specs=pl.BlockSpec((br, dout), lambda i: (_i32(i), _i32(i) * 0)),
        out_shape=jax.ShapeDtypeStruct((npr, dout), F32))


def _pad_edges(src, dst, n_in, n_out):
    e = src.shape[0]
    e_pad = _round_up(e, NW * C)
    pad = e_pad - e
    i = jnp.arange(pad, dtype=I32)
    src_p = jnp.concatenate([src, i % jnp.int32(n_in)])
    dst_p = jnp.concatenate([dst, jnp.int32(n_out) + (i % 64)])
    return src_p, dst_p, e_pad


def _sinus_row(t, dim):
    half = dim // 2
    cst = math.log(10000.0) / (half - 1)
    freqs = jnp.exp(jnp.arange(half, dtype=F32) * (-cst))
    e = t[0].astype(F32) * freqs
    return jnp.concatenate([jnp.sin(e), jnp.cos(e)])


def kernel(noised_data, t, edge_index, W_in, b_in, W0, b0, W1, b1, W2, b2,
           p_w, p_b, gamma, beta, W_fc, b_fc):
    from jax._src import config as _jcfg
    with _jcfg.enable_x64(False):
        return _kernel_impl(noised_data, t, edge_index, W_in, b_in, W0, b0,
                            W1, b1, W2, b2, p_w, p_b, gamma, beta, W_fc, b_fc)


def _kernel_impl(noised_data, t, edge_index, W_in, b_in, W0, b0, W1, b1, W2,
                 b2, p_w, p_b, gamma, beta, W_fc, b_fc):
    n = noised_data.shape[1]
    kk = n // 2
    d = W0.shape[1]
    ei = edge_index.astype(I32)
    src0, dst0 = ei[0], ei[1]
    e = src0.shape[0]

    pad_n = _round_up(n + 64, 1024)
    k_pad = _round_up(kk, NTILE * C)
    pad_k = k_pad
    assert kk + 1024 <= pad_k
    br_n = pad_n // 8
    br_k = pad_k // 8

    src0p, dst0p, e_pad = _pad_edges(src0, dst0, n, n)

    dp = _deg_pass(pad_n, e_pad)(dst0p)
    r0p = lax.rsqrt(dp[:pad_n] + dp[pad_n:] + 1.0)[:, None]

    xp = jnp.pad(noised_data[0], ((0, pad_n - n), (0, 0)))
    temb = _sinus_row(t, d)
    c0 = (temb @ W0[W_in.shape[1]:])[None]
    Hs0 = _front(pad_n, br_n, W_in.shape[0])(
        xp, W_in, b_in[None], W0[: W_in.shape[1]], c0, r0p)

    S = _edge_pass(pad_n, e_pad, d)(Hs0, src0p, dst0p)
    y, Q, A = _post0(pad_n, br_n)(
        S, Hs0, r0p, b0[None], p_w, p_b[None].astype(F32), W1, W2)

    yflat = y.reshape(pad_n)
    thr = _topk_thresh(pad_n, n, kk)(y.reshape(pad_n // 128, 128))
    idxfull = _compact(pad_n, n, kk, k_pad)(yflat, thr)
    idx32 = idxfull[:kk]

    tbl = kk * kk + e_pad
    s1a, d1a, keya, tbla = _map_edges(pad_n, k_pad, e_pad, kk, tbl)(
        idxfull, src0p, dst0p)
    s1p, d1p, dego1 = _finish_edges(pad_k, e_pad, kk)(s1a, d1a, keya, tbla)
    r1p = lax.rsqrt(dego1[:pad_k] + dego1[pad_k:] + 1.0)[:, None]

    H1 = _gather_rows(k_pad, d)(Q, idxfull)
    Hs1 = H1 * r1p
    S1 = _edge_pass(pad_k, e_pad, d)(Hs1, s1p, d1p)
    B = _post1(pad_k, br_k)(S1, Hs1, r1p, b1[None], W2)

    usrc, udst, ue_pad = _pad_edges(jnp.arange(kk, dtype=I32), idx32, kk, n)
    SU = _edge_pass(pad_n, ue_pad, d)(B, usrc, udst)
    Hs2 = (SU[0] + SU[1] + A) * r0p

    S2 = _edge_pass(pad_n, e_pad, d)(Hs2, src0p, dst0p)
    out = _post2(pad_n, br_n, W_fc.shape[1])(
        S2, Hs2, r0p, b2[None], gamma[None], beta[None], W_fc, b_fc[None])
    return out[:n][None]


# speedup vs baseline: 1.1918x; 1.1868x over previous
import functools
import math

import jax
import jax.numpy as jnp
from jax import lax
from jax.experimental import pallas as pl
from jax.experimental.pallas import tpu as pltpu
from jax.experimental.pallas import tpu_sc as plsc

F32 = jnp.float32
I32 = jnp.int32
NW = 32
NTILE = 16
C = 128


def _round_up(x, m):
    return (x + m - 1) // m * m


CO = 64


@functools.cache
def _edge_pass(n_out_pad, e_pad, d):
    w_per = e_pad // (NW * C)
    mesh = plsc.VectorSubcoreMesh(core_axis_name="c", subcore_axis_name="s")
    rpt = n_out_pad // NTILE
    assert rpt % CO == 0

    @functools.partial(
        pl.kernel,
        name="edge_pass",
        out_type=jax.ShapeDtypeStruct((2, n_out_pad, d), F32),
        mesh=mesh,
        scratch_types=[
            pltpu.VMEM((C,), I32),
            pltpu.VMEM((C,), I32),
            pltpu.VMEM((C,), I32),
            pltpu.VMEM((C,), I32),
            pltpu.VMEM((C, d), F32),
            pltpu.VMEM((C, d), F32),
            pltpu.VMEM((CO, d), F32),
            pltpu.VMEM_SHARED((n_out_pad, d), F32),
            pltpu.SemaphoreType.DMA,
            pltpu.SemaphoreType.DMA,
        ],
    )
    def k(hs, srcp, dstp, out, src_v0, dst_v0, src_v1, dst_v1, rows_v0,
          rows_v1, stg_v, acc, sem0, sem1):
        c = lax.axis_index("c")
        s = lax.axis_index("s")
        wid = c * NTILE + s
        src_v = (src_v0, src_v1)
        dst_v = (dst_v0, dst_v1)
        rows_v = (rows_v0, rows_v1)
        sem = (sem0, sem1)

        def zfill(i, carry):
            stg_v[i // jnp.int32(d // 16),
                  pl.ds((i % jnp.int32(d // 16)) * 16, 16)] = (
                      jnp.zeros((16,), F32))
            return carry
        lax.fori_loop(jnp.int32(0), jnp.int32(CO * d // 16), zfill,
                      jnp.int32(0))

        def zcp(i, carry):
            pltpu.sync_copy(stg_v, acc.at[pl.ds(s * rpt + i * jnp.int32(CO),
                                                CO)])
            return carry
        lax.fori_loop(jnp.int32(0), jnp.int32(rpt // CO), zcp, jnp.int32(0))
        plsc.subcore_barrier()

        tbase = wid * jnp.int32(w_per * C)

        def ldidx(w, b):
            pltpu.sync_copy(srcp.at[pl.ds(tbase + w * jnp.int32(C), C)],
                            src_v[b])
            pltpu.sync_copy(dstp.at[pl.ds(tbase + w * jnp.int32(C), C)],
                            dst_v[b])

        def stage(w, b):
            bn = 1 - b
            ldidx(w + jnp.int32(1), bn)
            pltpu.async_copy(hs.at[src_v[bn]], rows_v[bn], sem[bn])
            pltpu.make_async_copy(hs.at[src_v[b]], rows_v[b], sem[b]).wait()
            pltpu.sync_copy(rows_v[b], acc.at[dst_v[b]], add=True)

        ldidx(jnp.int32(0), 0)
        pltpu.async_copy(hs.at[src_v[0]], rows_v[0], sem[0])

        def body(w2, carry):
            w = w2 * jnp.int32(2)
            stage(w, 0)
            stage(w + jnp.int32(1), 1)
            return carry

        lax.fori_loop(jnp.int32(0), jnp.int32((w_per - 1) // 2), body,
                      jnp.int32(0))
        if (w_per - 1) % 2 == 1:
            stage(jnp.int32(w_per - 2), (w_per - 2) % 2)
        bl = (w_per - 1) % 2
        pltpu.make_async_copy(hs.at[src_v[bl]], rows_v[bl], sem[bl]).wait()
        pltpu.sync_copy(rows_v[bl], acc.at[dst_v[bl]], add=True)
        plsc.subcore_barrier()

        def ocp(i, carry):
            off = s * rpt + i * jnp.int32(CO)
            pltpu.sync_copy(acc.at[pl.ds(off, CO)], stg_v)
            pltpu.sync_copy(stg_v, out.at[c, pl.ds(off, CO)])
            return carry
        lax.fori_loop(jnp.int32(0), jnp.int32(rpt // CO), ocp, jnp.int32(0))

    return k


@functools.cache
def _deg_pass(n_out_pad, e_pad):
    w_per = e_pad // (NW * C)
    mesh = plsc.VectorSubcoreMesh(core_axis_name="c", subcore_axis_name="s")
    rpt = n_out_pad // NTILE

    @functools.partial(
        pl.kernel,
        name="deg_pass",
        out_type=jax.ShapeDtypeStruct((2 * n_out_pad,), F32),
        mesh=mesh,
        scratch_types=[
            pltpu.VMEM((C,), I32),
            pltpu.VMEM((C,), F32),
            pltpu.VMEM((rpt,), F32),
            pltpu.VMEM_SHARED((n_out_pad,), F32),
        ],
    )
    def k(dstp, out, dst_v, ones_v, stg_v, acc):
        c = lax.axis_index("c")
        s = lax.axis_index("s")
        wid = c * NTILE + s
        for i in range(C // 16):
            ones_v[pl.ds(i * 16, 16)] = jnp.full((16,), 1.0, F32)

        def zfill(i, carry):
            stg_v[pl.ds(i * 16, 16)] = jnp.zeros((16,), F32)
            return carry
        lax.fori_loop(jnp.int32(0), jnp.int32(rpt // 16), zfill, jnp.int32(0))
        pltpu.sync_copy(stg_v, acc.at[pl.ds(s * rpt, rpt)])
        plsc.subcore_barrier()

        def body(w, carry):
            base = (wid * jnp.int32(w_per) + w) * jnp.int32(C)
            pltpu.sync_copy(dstp.at[pl.ds(base, C)], dst_v)
            pltpu.sync_copy(ones_v, acc.at[dst_v], add=True)
            return carry

        lax.fori_loop(jnp.int32(0), jnp.int32(w_per), body, jnp.int32(0))
        plsc.subcore_barrier()
        pltpu.sync_copy(acc.at[pl.ds(s * rpt, rpt)], stg_v)
        pltpu.sync_copy(
            stg_v, out.at[pl.ds(c * jnp.int32(n_out_pad) + s * rpt, rpt)])

    return k


def _iota16():
    return lax.iota(I32, 16)


@functools.cache
def _map_edges(n_pad, k_pad, e_pad, kk, tbl):
    w_map = k_pad // (NTILE * C)
    w_per = e_pad // (NW * C)
    mesh = plsc.VectorSubcoreMesh(core_axis_name="c", subcore_axis_name="s")
    rpt = n_pad // NTILE
    ktrash = kk * kk

    @functools.partial(
        pl.kernel,
        name="map_edges",
        out_type=(jax.ShapeDtypeStruct((e_pad,), I32),
                  jax.ShapeDtypeStruct((e_pad,), I32),
                  jax.ShapeDtypeStruct((e_pad,), I32),
                  jax.ShapeDtypeStruct((tbl,), I32)),
        mesh=mesh,
        scratch_types=[
            pltpu.VMEM((C,), I32),
            pltpu.VMEM((C,), I32),
            pltpu.VMEM((C,), I32),
            pltpu.VMEM((C,), I32),
            pltpu.VMEM((C,), I32),
            pltpu.VMEM((C,), I32),
            pltpu.VMEM((rpt,), I32),
            pltpu.VMEM_SHARED((n_pad,), I32),
        ],
    )
    def k(idxp, srcp, dstp, s1o, d1o, keyo, tblo,
          a_v, b_v, s_v, d_v, key_v, eid_v, stg_v, newid):
        c = lax.axis_index("c")
        s = lax.axis_index("s")
        wid = c * NTILE + s

        def ifill(i, carry):
            stg_v[pl.ds(i * 16, 16)] = jnp.full((16,), -1, I32)
            return carry
        lax.fori_loop(jnp.int32(0), jnp.int32(rpt // 16), ifill, jnp.int32(0))
        pltpu.sync_copy(stg_v, newid.at[pl.ds(s * rpt, rpt)])
        plsc.subcore_barrier()

        def mbody(w, carry):
            base = (s * jnp.int32(w_map) + w) * jnp.int32(C)
            pltpu.sync_copy(idxp.at[pl.ds(base, C)], a_v)
            for j in range(C // 16):
                eid_v[pl.ds(j * 16, 16)] = base + jnp.int32(j * 16) + _iota16()
            pltpu.sync_copy(eid_v, newid.at[a_v])
            return carry
        lax.fori_loop(jnp.int32(0), jnp.int32(w_map), mbody, jnp.int32(0))
        plsc.subcore_barrier()

        def body(w, carry):
            base = (wid * jnp.int32(w_per) + w) * jnp.int32(C)
            pltpu.sync_copy(srcp.at[pl.ds(base, C)], a_v)
            pltpu.sync_copy(dstp.at[pl.ds(base, C)], b_v)
            pltpu.sync_copy(newid.at[a_v], s_v)
            pltpu.sync_copy(newid.at[b_v], d_v)
            for j in range(C // 16):
                sl = pl.ds(j * 16, 16)
                s16 = s_v[sl]
                d16 = d_v[sl]
                eid16 = base + jnp.int32(j * 16) + _iota16()
                m16 = (s16 >= 0) & (d16 >= 0)
                key16 = jnp.where(m16, s16 * jnp.int32(kk) + d16,
                                  jnp.int32(ktrash) + eid16)
                key_v[sl] = key16
                eid_v[sl] = eid16
            pltpu.sync_copy(s_v, s1o.at[pl.ds(base, C)])
            pltpu.sync_copy(d_v, d1o.at[pl.ds(base, C)])
            pltpu.sync_copy(key_v, keyo.at[pl.ds(base, C)])
            pltpu.sync_copy(eid_v, tblo.at[key_v])
            return carry
        lax.fori_loop(jnp.int32(0), jnp.int32(w_per), body, jnp.int32(0))

    return k


@functools.cache
def _finish_edges(pad_k, e_pad, kk):
    w_per = e_pad // (NW * C)
    mesh = plsc.VectorSubcoreMesh(core_axis_name="c", subcore_axis_name="s")
    rpt = pad_k // NTILE

    @functools.partial(
        pl.kernel,
        name="finish_edges",
        out_type=(jax.ShapeDtypeStruct((e_pad,), I32),
                  jax.ShapeDtypeStruct((e_pad,), I32),
                  jax.ShapeDtypeStruct((2 * pad_k,), F32)),
        mesh=mesh,
        scratch_types=[
            pltpu.VMEM((C,), I32),
            pltpu.VMEM((C,), I32),
            pltpu.VMEM((C,), I32),
            pltpu.VMEM((C,), I32),
            pltpu.VMEM((C,), I32),
            pltpu.VMEM((C,), I32),
            pltpu.VMEM((C,), F32),
            pltpu.VMEM((rpt,), F32),
            pltpu.VMEM_SHARED((pad_k,), F32),
            pltpu.SemaphoreType.DMA,
        ],
    )
    def k(s1, d1, key, tbl, srcfo, dstfo, dego,
          s_v, d_v, key_v, t_v, sf_v, df_v, ones_v, stg_v, acc, sem):
        c = lax.axis_index("c")
        s = lax.axis_index("s")
        wid = c * NTILE + s
        for i in range(C // 16):
            ones_v[pl.ds(i * 16, 16)] = jnp.full((16,), 1.0, F32)

        def zfill(i, carry):
            stg_v[pl.ds(i * 16, 16)] = jnp.zeros((16,), F32)
            return carry
        lax.fori_loop(jnp.int32(0), jnp.int32(rpt // 16), zfill, jnp.int32(0))
        pltpu.sync_copy(stg_v, acc.at[pl.ds(s * rpt, rpt)])
        plsc.subcore_barrier()

        def body(w, carry):
            base = (wid * jnp.int32(w_per) + w) * jnp.int32(C)
            pltpu.sync_copy(s1.at[pl.ds(base, C)], s_v)
            pltpu.sync_copy(d1.at[pl.ds(base, C)], d_v)
            pltpu.sync_copy(key.at[pl.ds(base, C)], key_v)
            pltpu.async_copy(tbl.at[key_v], t_v, sem).wait()
            for j in range(C // 16):
                sl = pl.ds(j * 16, 16)
                s16 = s_v[sl]
                d16 = d_v[sl]
                eid16 = base + jnp.int32(j * 16) + _iota16()
                ok = (s16 >= 0) & (d16 >= 0) & (t_v[sl] == eid16)
                sf_v[sl] = jnp.where(ok, s16, eid16 & 4095)
                df_v[sl] = jnp.where(ok, d16,
                                     jnp.int32(kk) + (eid16 & 1023))
            pltpu.sync_copy(sf_v, srcfo.at[pl.ds(base, C)])
            pltpu.sync_copy(df_v, dstfo.at[pl.ds(base, C)])
            pltpu.sync_copy(ones_v, acc.at[df_v], add=True)
            return carry
        lax.fori_loop(jnp.int32(0), jnp.int32(w_per), body, jnp.int32(0))
        plsc.subcore_barrier()
        pltpu.sync_copy(acc.at[pl.ds(s * rpt, rpt)], stg_v)
        pltpu.sync_copy(
            stg_v, dego.at[pl.ds(c * jnp.int32(pad_k) + s * rpt, rpt)])

    return k


@functools.cache
def _topk_thresh(npad, n, kk):
    nb = npad // 128

    def body(y_ref, thr_ref):
        y = y_ref[...]
        ib = pltpu.bitcast(y, jnp.int32)
        key = ib ^ ((ib >> 31) & jnp.int32(0x7FFFFFFF))
        rowi = lax.broadcasted_iota(jnp.int32, (nb, 128), 0)
        lanei = lax.broadcasted_iota(jnp.int32, (nb, 128), 1)
        key = jnp.where(rowi * 128 + lanei < n, key, jnp.int32(-2**31))

        v = jnp.int32(0)
        for b in range(31, -1, -1):
            vc = v | jnp.int32(-2**31 if b == 31 else 1 << b)
            cand = vc ^ jnp.int32(-2**31)
            cnt = jnp.sum((key >= cand).astype(F32), dtype=F32)
            v = jnp.where(cnt >= F32(kk), vc, v)
        t = v ^ jnp.int32(-2**31)
        thr_ref[...] = jnp.full((8, 128), t, jnp.int32)

    return pl.pallas_call(
        body, out_shape=jax.ShapeDtypeStruct((8, 128), jnp.int32))


@functools.cache
def _compact(npad, n, kk, k_pad):
    cpt = npad // NTILE
    nv = cpt // 16
    mesh = plsc.VectorSubcoreMesh(core_axis_name="c", subcore_axis_name="s")

    @functools.partial(
        pl.kernel,
        name="topk_compact",
        out_type=jax.ShapeDtypeStruct((k_pad + 2048,), I32),
        mesh=mesh,
        compiler_params=pltpu.CompilerParams(needs_layout_passes=False),
        scratch_types=[
            pltpu.VMEM((cpt,), F32),
            pltpu.VMEM((8, 128), I32),
            pltpu.VMEM((16,), I32),
            pltpu.VMEM((16,), I32),
            pltpu.VMEM((32,), I32),
            pltpu.VMEM((16,), I32),
            pltpu.VMEM((16,), I32),
            pltpu.VMEM((1152,), I32),
            pltpu.VMEM_SHARED((32,), I32),
        ],
    )
    def k(yh, thrh, idxo, y_v, thr_vm, si_v, cv_v, call_v, pos_v, val_v,
          pat_v, counts_sp):
        c = lax.axis_index("c")
        s = lax.axis_index("s")

        @pl.when(c == 0)
        def _():
            pltpu.sync_copy(yh.at[pl.ds(s * jnp.int32(cpt), cpt)], y_v)
            pltpu.sync_copy(thrh, thr_vm)
            thr = thr_vm[0, pl.ds(0, 16)]

            def mono(y16):
                ib = plsc.bitcast(y16, I32)
                return ib ^ ((ib >> 31) & jnp.int32(0x7FFFFFFF))

            def masks(j):
                y16 = y_v[pl.ds(j * jnp.int32(16), 16)]
                u = mono(y16)
                node = s * jnp.int32(cpt) + j * jnp.int32(16) + _iota16()
                msk = node < jnp.int32(n)
                return node, msk & (u > thr), msk & (u == thr)

            def p1(j, carry):
                csel, ctie = carry
                _, sel, tie = masks(j)
                return (csel + plsc.all_reduce_population_count(sel),
                        ctie + plsc.all_reduce_population_count(tie))

            z16 = jnp.zeros((16,), I32)
            csel, ctie = lax.fori_loop(jnp.int32(0), jnp.int32(nv), p1, (z16, z16))

            si_v[...] = jnp.full((16,), 0, I32) + s
            cv_v[...] = csel
            pltpu.sync_copy(cv_v, counts_sp.at[si_v])
            si_v[...] = jnp.full((16,), 16, I32) + s
            cv_v[...] = ctie
            pltpu.sync_copy(cv_v, counts_sp.at[si_v])
            plsc.subcore_barrier()
            pltpu.sync_copy(counts_sp, call_v)
            selc = call_v[pl.ds(0, 16)]
            tiec = call_v[pl.ds(16, 16)]
            inc = plsc.cumsum(selc)
            tin = plsc.cumsum(tiec)
            lane_s = jnp.full((16,), 0, I32) + s
            cv_v[...] = inc - selc
            soff = plsc.load_gather(cv_v, [lane_s])
            cv_v[...] = tin - tiec
            toff = plsc.load_gather(cv_v, [lane_s])
            cv_v[...] = inc
            g_tot = plsc.load_gather(cv_v, [jnp.full((16,), 15, I32)])
            rneed = jnp.full((16,), kk, I32) - g_tot

            def p2(j, carry):
                so, to = carry
                node, sel, tie = masks(j)
                cs = plsc.cumsum(jnp.where(sel, jnp.int32(1), jnp.int32(0)))
                ct = plsc.cumsum(jnp.where(tie, jnp.int32(1), jnp.int32(0)))
                grank = to + ct - 1
                acc = tie & (grank < rneed)
                trash = jnp.full((16,), k_pad, I32) + s * jnp.int32(64) + j
                pos = jnp.where(sel, so + cs - 1,
                                jnp.where(acc, g_tot + grank, trash))
                pos_v[...] = pos
                val_v[...] = node
                pltpu.sync_copy(val_v, idxo.at[pos_v])
                return (so + plsc.all_reduce_population_count(sel),
                        to + plsc.all_reduce_population_count(tie))

            lax.fori_loop(jnp.int32(0), jnp.int32(nv), p2, (soff, toff))

            @pl.when(s == 0)
            def _():
                def pf(i, carry):
                    t16 = i * jnp.int32(16) + _iota16()
                    pat_v[pl.ds(i * jnp.int32(16), 16)] = jnp.int32(n + 64) + (t16 & 63)
                    return carry
                lax.fori_loop(jnp.int32(0), jnp.int32((k_pad - kk) // 16 + 1), pf, jnp.int32(0))
                pltpu.sync_copy(pat_v.at[pl.ds(0, k_pad - kk)],
                                idxo.at[pl.ds(kk, k_pad - kk)])

    return k


@functools.cache
def _gather_rows(m_rows, d):
    cw = 64
    w_per = m_rows // (NW * cw)
    mesh = plsc.VectorSubcoreMesh(core_axis_name="c", subcore_axis_name="s")

    @functools.partial(
        pl.kernel,
        name="gather_rows",
        out_type=jax.ShapeDtypeStruct((m_rows, d), F32),
        mesh=mesh,
        scratch_types=[
            pltpu.VMEM((cw,), I32),
            pltpu.VMEM((cw, d), F32),
            pltpu.SemaphoreType.DMA,
        ],
    )
    def k(hs, idxh, out, i_v, rows_v, sem):
        c = lax.axis_index("c")
        s = lax.axis_index("s")
        wid = c * NTILE + s

        def body(w, carry):
            base = (wid * jnp.int32(w_per) + w) * jnp.int32(cw)
            pltpu.sync_copy(idxh.at[pl.ds(base, cw)], i_v)
            pltpu.async_copy(hs.at[i_v], rows_v, sem).wait()
            pltpu.sync_copy(rows_v, out.at[pl.ds(base, cw)])
            return carry

        lax.fori_loop(jnp.int32(0), jnp.int32(w_per), body, jnp.int32(0))

    return k


def _i32(v):
    return lax.convert_element_type(v, jnp.int32)


def _rep(shape):
    return lambda i: tuple(_i32(i) * 0 for _ in shape)


@functools.cache
def _front(npr, br, din):
    def body(x_ref, wi_ref, bi_ref, w0_ref, c0_ref, r_ref, o_ref):
        h = jnp.dot(x_ref[...], wi_ref[...],
                    preferred_element_type=F32) + bi_ref[...]
        h = jnp.dot(h, w0_ref[...], preferred_element_type=F32) + c0_ref[...]
        o_ref[...] = h * r_ref[...]

    return pl.pallas_call(
        body,
        grid=(npr // br,),
        in_specs=[pl.BlockSpec((br, din), lambda i: (_i32(i), _i32(i) * 0)),
                  pl.BlockSpec((din, 128), _rep((0, 0))),
                  pl.BlockSpec((1, 128), _rep((0, 0))),
                  pl.BlockSpec((128, 128), _rep((0, 0))),
                  pl.BlockSpec((1, 128), _rep((0, 0))),
                  pl.BlockSpec((br, 1), lambda i: (_i32(i), _i32(i) * 0))],
        out_specs=pl.BlockSpec((br, 128), lambda i: (_i32(i), _i32(i) * 0)),
        out_shape=jax.ShapeDtypeStruct((npr, 128), F32))


@functools.cache
def _post0(npr, br):
    def body(s_ref, hs_ref, r_ref, b_ref, pw_ref, pb_ref, w1_ref, w2_ref,
             y_ref, q_ref, a_ref):
        s = s_ref[...]
        h = jax.nn.relu(r_ref[...] * (s[0] + s[1] + hs_ref[...]) + b_ref[...])
        y = jnp.dot(h, pw_ref[...], preferred_element_type=F32) + pb_ref[0, 0]
        g = h * (1.0 / (1.0 + jnp.exp(-y)))
        y_ref[...] = y
        q_ref[...] = jnp.dot(g, w1_ref[...], preferred_element_type=F32)
        a_ref[...] = jnp.dot(h, w2_ref[...], preferred_element_type=F32)

    return pl.pallas_call(
        body,
        grid=(npr // br,),
        in_specs=[pl.BlockSpec((2, br, 128), lambda i: (_i32(i) * 0, _i32(i), _i32(i) * 0)),
                  pl.BlockSpec((br, 128), lambda i: (_i32(i), _i32(i) * 0)),
                  pl.BlockSpec((br, 1), lambda i: (_i32(i), _i32(i) * 0)),
                  pl.BlockSpec((1, 128), _rep((0, 0))),
                  pl.BlockSpec((128, 1), _rep((0, 0))),
                  pl.BlockSpec(memory_space=pltpu.SMEM),
                  pl.BlockSpec((128, 128), _rep((0, 0))),
                  pl.BlockSpec((128, 128), _rep((0, 0)))],
        out_specs=[pl.BlockSpec((br, 1), lambda i: (_i32(i), _i32(i) * 0)),
                   pl.BlockSpec((br, 128), lambda i: (_i32(i), _i32(i) * 0)),
                   pl.BlockSpec((br, 128), lambda i: (_i32(i), _i32(i) * 0))],
        out_shape=[jax.ShapeDtypeStruct((npr, 1), F32),
                   jax.ShapeDtypeStruct((npr, 128), F32),
                   jax.ShapeDtypeStruct((npr, 128), F32)])


@functools.cache
def _post1(npr, br):
    def body(s_ref, hs_ref, r_ref, b_ref, w2_ref, o_ref):
        s = s_ref[...]
        h = jax.nn.relu(r_ref[...] * (s[0] + s[1] + hs_ref[...]) + b_ref[...])
        o_ref[...] = jnp.dot(h, w2_ref[...], preferred_element_type=F32)

    return pl.pallas_call(
        body,
        grid=(npr // br,),
        in_specs=[pl.BlockSpec((2, br, 128), lambda i: (_i32(i) * 0, _i32(i), _i32(i) * 0)),
                  pl.BlockSpec((br, 128), lambda i: (_i32(i), _i32(i) * 0)),
                  pl.BlockSpec((br, 1), lambda i: (_i32(i), _i32(i) * 0)),
                  pl.BlockSpec((1, 128), _rep((0, 0))),
                  pl.BlockSpec((128, 128), _rep((0, 0)))],
        out_specs=pl.BlockSpec((br, 128), lambda i: (_i32(i), _i32(i) * 0)),
        out_shape=jax.ShapeDtypeStruct((npr, 128), F32))


@functools.cache
def _post2(npr, br, dout):
    def body(s_ref, hs_ref, r_ref, b_ref, g_ref, be_ref, wf_ref, bf_ref,
             o_ref):
        s = s_ref[...]
        h = jax.nn.relu(r_ref[...] * (s[0] + s[1] + hs_ref[...]) + b_ref[...])
        mu = jnp.mean(h, axis=-1, keepdims=True)
        var = jnp.mean((h - mu) ** 2, axis=-1, keepdims=True)
        ln = (h - mu) / jnp.sqrt(var + 1e-5) * g_ref[...] + be_ref[...]
        o_ref[...] = jnp.dot(ln, wf_ref[...],
                             preferred_element_type=F32) + bf_ref[...]

    return pl.pallas_call(
        body,
        grid=(npr // br,),
        in_specs=[pl.BlockSpec((2, br, 128), lambda i: (_i32(i) * 0, _i32(i), _i32(i) * 0)),
                  pl.BlockSpec((br, 128), lambda i: (_i32(i), _i32(i) * 0)),
                  pl.BlockSpec((br, 1), lambda i: (_i32(i), _i32(i) * 0)),
                  pl.BlockSpec((1, 128), _rep((0, 0))),
                  pl.BlockSpec((1, 128), _rep((0, 0))),
                  pl.BlockSpec((1, 128), _rep((0, 0))),
                  pl.BlockSpec((128, dout), _rep((0, 0))),
                  pl.BlockSpec((1, dout), _rep((0, 0)))],
        out_specs=pl.BlockSpec((br, dout), lambda i: (_i32(i), _i32(i) * 0)),
        out_shape=jax.ShapeDtypeStruct((npr, dout), F32))


def _pad_edges(src, dst, n_in, n_out):
    e = src.shape[0]
    e_pad = _round_up(e, NW * C)
    pad = e_pad - e
    i = jnp.arange(pad, dtype=I32)
    src_p = jnp.concatenate([src, i % jnp.int32(n_in)])
    dst_p = jnp.concatenate([dst, jnp.int32(n_out) + (i % 64)])
    return src_p, dst_p, e_pad


def _sinus_row(t, dim):
    half = dim // 2
    cst = math.log(10000.0) / (half - 1)
    freqs = jnp.exp(jnp.arange(half, dtype=F32) * (-cst))
    e = t[0].astype(F32) * freqs
    return jnp.concatenate([jnp.sin(e), jnp.cos(e)])


def kernel(noised_data, t, edge_index, W_in, b_in, W0, b0, W1, b1, W2, b2,
           p_w, p_b, gamma, beta, W_fc, b_fc):
    from jax._src import config as _jcfg
    with _jcfg.enable_x64(False):
        return _kernel_impl(noised_data, t, edge_index, W_in, b_in, W0, b0,
                            W1, b1, W2, b2, p_w, p_b, gamma, beta, W_fc, b_fc)


def _kernel_impl(noised_data, t, edge_index, W_in, b_in, W0, b0, W1, b1, W2,
                 b2, p_w, p_b, gamma, beta, W_fc, b_fc):
    n = noised_data.shape[1]
    kk = n // 2
    d = W0.shape[1]
    ei = edge_index.astype(I32)
    src0, dst0 = ei[0], ei[1]
    e = src0.shape[0]

    pad_n = _round_up(n + 64, 1024)
    k_pad = _round_up(kk, NTILE * C)
    pad_k = k_pad
    assert kk + 1024 <= pad_k
    br_n = pad_n // 8
    br_k = pad_k // 8

    src0p, dst0p, e_pad = _pad_edges(src0, dst0, n, n)

    dp = _deg_pass(pad_n, e_pad)(dst0p)
    r0p = lax.rsqrt(dp[:pad_n] + dp[pad_n:] + 1.0)[:, None]

    xp = jnp.pad(noised_data[0], ((0, pad_n - n), (0, 0)))
    temb = _sinus_row(t, d)
    c0 = (temb @ W0[W_in.shape[1]:])[None]
    Hs0 = _front(pad_n, br_n, W_in.shape[0])(
        xp, W_in, b_in[None], W0[: W_in.shape[1]], c0, r0p)

    S = _edge_pass(pad_n, e_pad, d)(Hs0, src0p, dst0p)
    y, Q, A = _post0(pad_n, br_n)(
        S, Hs0, r0p, b0[None], p_w, p_b[None].astype(F32), W1, W2)

    yflat = y.reshape(pad_n)
    thr = _topk_thresh(pad_n, n, kk)(y.reshape(pad_n // 128, 128))
    idxfull = _compact(pad_n, n, kk, k_pad)(yflat, thr)
    idx32 = idxfull[:kk]

    tbl = kk * kk + e_pad
    s1a, d1a, keya, tbla = _map_edges(pad_n, k_pad, e_pad, kk, tbl)(
        idxfull, src0p, dst0p)
    s1p, d1p, dego1 = _finish_edges(pad_k, e_pad, kk)(s1a, d1a, keya, tbla)
    r1p = lax.rsqrt(dego1[:pad_k] + dego1[pad_k:] + 1.0)[:, None]

    H1 = _gather_rows(k_pad, d)(Q, idxfull)
    Hs1 = H1 * r1p
    S1 = _edge_pass(pad_k, e_pad, d)(Hs1, s1p, d1p)
    B = _post1(pad_k, br_k)(S1, Hs1, r1p, b1[None], W2)

    usrc, udst, ue_pad = _pad_edges(jnp.arange(kk, dtype=I32), idx32, kk, n)
    SU = _edge_pass(pad_n, ue_pad, d)(B, usrc, udst)
    Hs2 = (SU[0] + SU[1] + A) * r0p

    S2 = _edge_pass(pad_n, e_pad, d)(Hs2, src0p, dst0p)
    out = _post2(pad_n, br_n, W_fc.shape[1])(
        S2, Hs2, r0p, b2[None], gamma[None], beta[None], W_fc, b_fc[None])
    return out[:n][None]
